# baseline (speedup 1.0000x reference)
import jax
import jax.numpy as jnp
from jax import lax
from jax.experimental import pallas as pl
from jax.experimental.pallas import tpu as pltpu
from jax.experimental.pallas import tpu_sc as plsc

HIDDEN = 64
N_NODES = 50000
N_EDGES = 800000

NC = 2
NS = 16
NW = NC * NS
L = 16

EB_TOT = N_EDGES // 128
EB_PER_W = 196
E_PAIRS = EB_PER_W // 4

N_PAD = 50048
NB_TOT = N_PAD // 128
NB_PER_W = 14
N_PAIRS = NB_PER_W // 2

R0, RB, RC, RD, RE = 119, 576, 360, 4, 60


def _sc_body(x_hbm, ea_hbm, t0_hbm, tb_hbm, tc_hbm, td_hbm, fe_hbm,
             oute_hbm, outn_hbm,
             t0v, tbv, tcv, tdv, fev, ebuf, xbuf, r0buf, r1buf, s0, s1):
    cid = lax.axis_index("c")
    sid = lax.axis_index("s")
    wid = sid * NC + cid
    lanes = lax.iota(jnp.int32, L)
    lanes3 = lanes * 3
    lanes9 = lanes * 9

    pltpu.sync_copy(t0_hbm, t0v)
    pltpu.sync_copy(tb_hbm, tbv)
    pltpu.sync_copy(tc_hbm, tcv)
    pltpu.sync_copy(td_hbm, tdv)
    pltpu.sync_copy(fe_hbm, fev)

    bufs = ((r0buf, s0), (r1buf, s1))

    node_b0 = jnp.minimum(wid * (NB_PER_W - 1), NB_TOT - NB_PER_W)

    def node_pair(i, carry):
        nb = node_b0 + i * 2
        pltpu.sync_copy(x_hbm.at[pl.ds(nb * 1152, 2304)], xbuf)
        for p, (rbuf, sem) in enumerate(bufs):
            @pl.when(i > 0)
            def _():
                pltpu.make_async_copy(
                    outn_hbm.at[:, pl.ds(0, 1)], rbuf.at[:, pl.ds(0, 1)],
                    sem).wait()

            def lgloop(lg, c2):
                soff = pl.multiple_of(lg * 16, 16)
                xv = lanes9 + (lg * 144 + p * 1152)
                xc = [plsc.load_gather(xbuf, [xv + j]) for j in range(9)]
                aa = xc[0] * 64
                ab = ((xc[1] * 12 + xc[2]) * 12 + xc[3]) * 64
                ac = ((xc[4] * 6 + xc[5]) * 6 + xc[6]) * 64
                ad = (xc[7] * 2 + xc[8]) * 64
                for t in range(64):
                    va = plsc.load_gather(t0v, [aa + t])
                    vb = plsc.load_gather(tbv, [ab + t])
                    vc = plsc.load_gather(tcv, [ac + t])
                    vd = plsc.load_gather(tdv, [ad + t])
                    rbuf[t // 8, 0, t % 8, pl.ds(soff, 16)] = (
                        (va + vb) + (vc + vd))
                return c2

            lax.fori_loop(0, 8, lgloop, 0)
            pltpu.async_copy(rbuf.at[:, pl.ds(0, 1)],
                             outn_hbm.at[:, pl.ds(nb + p, 1)], sem)
        return carry

    lax.fori_loop(0, N_PAIRS, node_pair, 0)
    for rbuf, sem in bufs:
        pltpu.make_async_copy(
            outn_hbm.at[:, pl.ds(0, 1)], rbuf.at[:, pl.ds(0, 1)], sem).wait()

    edge_b0 = jnp.minimum(wid * EB_PER_W, EB_TOT - EB_PER_W)

    def edge_pair(k, carry):
        bb = edge_b0 + k * 4
        pltpu.sync_copy(ea_hbm.at[pl.ds(bb * 384, 1536)], ebuf)
        for p, (rbuf, sem) in enumerate(bufs):
            @pl.when(k > 0)
            def _():
                pltpu.make_async_copy(
                    oute_hbm.at[:, pl.ds(0, 2)], rbuf, sem).wait()
            for b in range(2):
                def lgloop(lg, c2, b=b):
                    soff = pl.multiple_of(lg * 16, 16)
                    ev = lanes3 + (lg * 48 + (p * 2 + b) * 384)
                    e0 = plsc.load_gather(ebuf, [ev])
                    e1 = plsc.load_gather(ebuf, [ev + 1])
                    e2 = plsc.load_gather(ebuf, [ev + 2])
                    a0 = ((e0 * 6 + e1) * 2 + e2) * 64
                    for t in range(64):
                        vals = plsc.load_gather(fev, [a0 + t])
                        rbuf[t // 8, b, t % 8, pl.ds(soff, 16)] = vals
                    return c2

                lax.fori_loop(0, 8, lgloop, 0)
            pltpu.async_copy(rbuf, oute_hbm.at[:, pl.ds(bb + p * 2, 2)], sem)
        return carry

    lax.fori_loop(0, E_PAIRS, edge_pair, 0)
    for rbuf, sem in bufs:
        pltpu.make_async_copy(oute_hbm.at[:, pl.ds(0, 2)], rbuf, sem).wait()


_sc_call = pl.kernel(
    _sc_body,
    out_type=(
        jax.ShapeDtypeStruct((8, EB_TOT, 8, 128), jnp.float32),
        jax.ShapeDtypeStruct((8, NB_TOT, 8, 128), jnp.float32),
    ),
    mesh=plsc.VectorSubcoreMesh(core_axis_name="c", subcore_axis_name="s"),
    compiler_params=pltpu.CompilerParams(
        needs_layout_passes=False, use_tc_tiling_on_sc=False),
    scratch_types=[
        pltpu.VMEM((R0 * HIDDEN,), jnp.float32),
        pltpu.VMEM((RB * HIDDEN,), jnp.float32),
        pltpu.VMEM((RC * HIDDEN,), jnp.float32),
        pltpu.VMEM((RD * HIDDEN,), jnp.float32),
        pltpu.VMEM((RE * HIDDEN,), jnp.float32),
        pltpu.VMEM((512 * 3,), jnp.int32),
        pltpu.VMEM((256 * 9,), jnp.int32),
        pltpu.VMEM((8, 2, 8, 128), jnp.float32),
        pltpu.VMEM((8, 2, 8, 128), jnp.float32),
        pltpu.SemaphoreType.DMA,
        pltpu.SemaphoreType.DMA,
    ],
)


def kernel(x, edge_attr, atom_tables, bond_tables):
    x32 = x.astype(jnp.int32)
    xf = jnp.concatenate(
        [x32, jnp.zeros((N_PAD - N_NODES, 9), jnp.int32)]).reshape(-1)
    eaf = edge_attr.astype(jnp.int32).reshape(-1)
    t = [a.astype(jnp.float32) for a in atom_tables]
    bo = [a.astype(jnp.float32) for a in bond_tables]
    t0f = t[0].reshape(-1)
    tbf = (t[1][:, None, None] + t[2][None, :, None]
           + t[3][None, None, :]).reshape(-1)
    tcf = (t[4][:, None, None] + t[5][None, :, None]
           + t[6][None, None, :]).reshape(-1)
    tdf = (t[7][:, None] + t[8][None, :]).reshape(-1)
    fef = (bo[0][:, None, None] + bo[1][None, :, None]
           + bo[2][None, None, :]).reshape(-1)
    o4e, o4n = _sc_call(xf, eaf, t0f, tbf, tcf, tdf, fef)
    edge_emb = o4e.transpose(1, 3, 0, 2).reshape(N_EDGES, HIDDEN)
    node_emb = o4n.transpose(1, 3, 0, 2).reshape(N_PAD, HIDDEN)[:N_NODES]
    return (node_emb, edge_emb)


# speedup vs baseline: 1.3262x; 1.3262x over previous
import jax
import jax.numpy as jnp
from jax import lax
from jax.experimental import pallas as pl
from jax.experimental.pallas import tpu as pltpu
from jax.experimental.pallas import tpu_sc as plsc

HIDDEN = 64
N_NODES = 50000
N_EDGES = 800000

NC = 2
NS = 16
NW = NC * NS
L = 16

EB_TOT = N_EDGES // 128
EB_PER_W = 196
E_PAIRS = EB_PER_W // 4

N_PAD = 50048
NB_TOT = N_PAD // 128
NB_PER_W = 14
N_PAIRS = NB_PER_W // 2

R0, RB, RC, RD, RE = 119, 576, 360, 4, 60
STR = 65


def _padup(n):
    return (n + 127) // 128 * 128


S0, SB, SC_, SD, SE = (_padup(r * STR) for r in (R0, RB, RC, RD, RE))


def _sc_body(x_hbm, ea_hbm, t0_hbm, tb_hbm, tc_hbm, td_hbm, fe_hbm,
             oute_hbm, outn_hbm,
             t0v, tbv, tcv, tdv, fev, ebuf, xbuf, r0buf, r1buf, s0, s1):
    cid = lax.axis_index("c")
    sid = lax.axis_index("s")
    wid = sid * NC + cid
    lanes = lax.iota(jnp.int32, L)
    lanes3 = lanes * 3
    lanes9 = lanes * 9

    pltpu.sync_copy(t0_hbm, t0v)
    pltpu.sync_copy(tb_hbm, tbv)
    pltpu.sync_copy(tc_hbm, tcv)
    pltpu.sync_copy(td_hbm, tdv)
    pltpu.sync_copy(fe_hbm, fev)

    bufs = ((r0buf, s0), (r1buf, s1))

    node_b0 = jnp.minimum(wid * (NB_PER_W - 1), NB_TOT - NB_PER_W)

    def node_pair(i, carry):
        nb = node_b0 + i * 2
        pltpu.sync_copy(x_hbm.at[pl.ds(nb * 1152, 2304)], xbuf)
        for p, (rbuf, sem) in enumerate(bufs):
            @pl.when(i > 0)
            def _():
                pltpu.make_async_copy(
                    outn_hbm.at[:, pl.ds(0, 1)], rbuf.at[:, pl.ds(0, 1)],
                    sem).wait()

            def lgloop(lg, c2):
                soff = pl.multiple_of(lg * 16, 16)
                xv = lanes9 + (lg * 144 + p * 1152)
                xc = [plsc.load_gather(xbuf, [xv + j]) for j in range(9)]
                aa = xc[0] * STR
                ab = ((xc[1] * 12 + xc[2]) * 12 + xc[3]) * STR
                ac = ((xc[4] * 6 + xc[5]) * 6 + xc[6]) * STR
                ad = (xc[7] * 2 + xc[8]) * STR
                for t in range(64):
                    va = plsc.load_gather(t0v, [aa + t])
                    vb = plsc.load_gather(tbv, [ab + t])
                    vc = plsc.load_gather(tcv, [ac + t])
                    vd = plsc.load_gather(tdv, [ad + t])
                    rbuf[t // 8, 0, t % 8, pl.ds(soff, 16)] = (
                        (va + vb) + (vc + vd))
                return c2

            lax.fori_loop(0, 8, lgloop, 0)
            pltpu.async_copy(rbuf.at[:, pl.ds(0, 1)],
                             outn_hbm.at[:, pl.ds(nb + p, 1)], sem)
        return carry

    lax.fori_loop(0, N_PAIRS, node_pair, 0)
    for rbuf, sem in bufs:
        pltpu.make_async_copy(
            outn_hbm.at[:, pl.ds(0, 1)], rbuf.at[:, pl.ds(0, 1)], sem).wait()

    edge_b0 = jnp.minimum(wid * EB_PER_W, EB_TOT - EB_PER_W)

    def edge_pair(k, carry):
        bb = edge_b0 + k * 4
        pltpu.sync_copy(ea_hbm.at[pl.ds(bb * 384, 1536)], ebuf)
        for p, (rbuf, sem) in enumerate(bufs):
            @pl.when(k > 0)
            def _():
                pltpu.make_async_copy(
                    oute_hbm.at[:, pl.ds(0, 2)], rbuf, sem).wait()
            for b in range(2):
                def lgloop(lg, c2, b=b):
                    soff = pl.multiple_of(lg * 16, 16)
                    ev = lanes3 + (lg * 48 + (p * 2 + b) * 384)
                    e0 = plsc.load_gather(ebuf, [ev])
                    e1 = plsc.load_gather(ebuf, [ev + 1])
                    e2 = plsc.load_gather(ebuf, [ev + 2])
                    a0 = ((e0 * 6 + e1) * 2 + e2) * STR
                    for t in range(64):
                        vals = plsc.load_gather(fev, [a0 + t])
                        rbuf[t // 8, b, t % 8, pl.ds(soff, 16)] = vals
                    return c2

                lax.fori_loop(0, 8, lgloop, 0)
            pltpu.async_copy(rbuf, oute_hbm.at[:, pl.ds(bb + p * 2, 2)], sem)
        return carry

    lax.fori_loop(0, E_PAIRS, edge_pair, 0)
    for rbuf, sem in bufs:
        pltpu.make_async_copy(oute_hbm.at[:, pl.ds(0, 2)], rbuf, sem).wait()


_sc_call = pl.kernel(
    _sc_body,
    out_type=(
        jax.ShapeDtypeStruct((8, EB_TOT, 8, 128), jnp.float32),
        jax.ShapeDtypeStruct((8, NB_TOT, 8, 128), jnp.float32),
    ),
    mesh=plsc.VectorSubcoreMesh(core_axis_name="c", subcore_axis_name="s"),
    compiler_params=pltpu.CompilerParams(
        needs_layout_passes=False, use_tc_tiling_on_sc=True),
    scratch_types=[
        pltpu.VMEM((S0,), jnp.float32),
        pltpu.VMEM((SB,), jnp.float32),
        pltpu.VMEM((SC_,), jnp.float32),
        pltpu.VMEM((SD,), jnp.float32),
        pltpu.VMEM((SE,), jnp.float32),
        pltpu.VMEM((512 * 3,), jnp.int32),
        pltpu.VMEM((256 * 9,), jnp.int32),
        pltpu.VMEM((8, 2, 8, 128), jnp.float32),
        pltpu.VMEM((8, 2, 8, 128), jnp.float32),
        pltpu.SemaphoreType.DMA,
        pltpu.SemaphoreType.DMA,
    ],
)


def kernel(x, edge_attr, atom_tables, bond_tables):
    x32 = x.astype(jnp.int32)
    xf = jnp.concatenate(
        [x32, jnp.zeros((N_PAD - N_NODES, 9), jnp.int32)]).reshape(-1)
    eaf = edge_attr.astype(jnp.int32).reshape(-1)
    t = [a.astype(jnp.float32) for a in atom_tables]
    bo = [a.astype(jnp.float32) for a in bond_tables]
    def _flat(tbl2d, size):
        r = tbl2d.shape[0]
        f = jnp.pad(tbl2d, ((0, 0), (0, STR - HIDDEN))).reshape(-1)
        return jnp.pad(f, (0, size - r * STR))

    t0f = _flat(t[0], S0)
    tbf = _flat((t[1][:, None, None] + t[2][None, :, None]
                 + t[3][None, None, :]).reshape(RB, HIDDEN), SB)
    tcf = _flat((t[4][:, None, None] + t[5][None, :, None]
                 + t[6][None, None, :]).reshape(RC, HIDDEN), SC_)
    tdf = _flat((t[7][:, None] + t[8][None, :]).reshape(RD, HIDDEN), SD)
    fef = _flat((bo[0][:, None, None] + bo[1][None, :, None]
                 + bo[2][None, None, :]).reshape(RE, HIDDEN), SE)
    o4e, o4n = _sc_call(xf, eaf, t0f, tbf, tcf, tdf, fef)
    edge_emb = o4e.transpose(1, 3, 0, 2).reshape(N_EDGES, HIDDEN)
    node_emb = o4n.transpose(1, 3, 0, 2).reshape(N_PAD, HIDDEN)[:N_NODES]
    return (node_emb, edge_emb)


# speedup vs baseline: 7.0169x; 5.2909x over previous
import jax
import jax.numpy as jnp
from jax import lax
from jax.experimental import pallas as pl
from jax.experimental.pallas import tpu as pltpu
from jax.experimental.pallas import tpu_sc as plsc

HIDDEN = 64
N_NODES = 50000
N_EDGES = 800000

NC = 2
NS = 16
NW = NC * NS
L = 16

EB_TOT = N_EDGES // 128
EB_PER_W = 196
E_PAIRS = EB_PER_W // 4

N_PAD = 50048
NB_TOT = N_PAD // 128
NB_PER_W = 14
N_PAIRS = NB_PER_W // 2

R0, RB, RC, RD, RE = 119, 576, 360, 4, 60
STR = 65


def _padup(n):
    return (n + 127) // 128 * 128


S0, SB, SC_, SD, SE = (_padup(r * STR) for r in (R0, RB, RC, RD, RE))


def _sc_body(x_hbm, ea_hbm, t0_hbm, tb_hbm, tc_hbm, td_hbm, fe_hbm,
             oute_hbm, outn_hbm,
             t0v, tbv, tcv, tdv, fev, ebuf, xbuf, r0buf, r1buf, s0, s1, si):
    cid = lax.axis_index("c")
    sid = lax.axis_index("s")
    wid = sid * NC + cid

    pltpu.sync_copy(t0_hbm, t0v)
    pltpu.sync_copy(tb_hbm, tbv)
    pltpu.sync_copy(tc_hbm, tcv)
    pltpu.sync_copy(td_hbm, tdv)
    pltpu.sync_copy(fe_hbm, fev)

    bufs = ((r0buf, s0), (r1buf, s1))

    node_b0 = jnp.minimum(wid * (NB_PER_W - 1), NB_TOT - NB_PER_W)

    def node_pair(i, carry):
        nb = node_b0 + i * 2
        xds = [pltpu.async_copy(
                   x_hbm.at[pl.ds(j * N_PAD + nb * 128, 256)],
                   xbuf.at[pl.ds(j * 256, 256)], si) for j in range(9)]
        for d in xds:
            d.wait()
        for p, (rbuf, sem) in enumerate(bufs):
            @pl.when(i > 0)
            def _():
                pltpu.make_async_copy(
                    outn_hbm.at[:, pl.ds(0, 1)], rbuf.at[:, pl.ds(0, 1)],
                    sem).wait()

            def lgloop(lg, c2):
                soff = pl.multiple_of(lg * 16, 16)
                xc = [xbuf[pl.ds(soff + (j * 256 + p * 128), 16)]
                      for j in range(9)]
                aa = xc[0] * STR
                ab = ((xc[1] * 12 + xc[2]) * 12 + xc[3]) * STR
                ac = ((xc[4] * 6 + xc[5]) * 6 + xc[6]) * STR
                ad = (xc[7] * 2 + xc[8]) * STR
                for t in range(64):
                    va = plsc.load_gather(t0v, [aa + t])
                    vb = plsc.load_gather(tbv, [ab + t])
                    vc = plsc.load_gather(tcv, [ac + t])
                    vd = plsc.load_gather(tdv, [ad + t])
                    rbuf[t // 8, 0, t % 8, pl.ds(soff, 16)] = (
                        (va + vb) + (vc + vd))
                return c2

            lax.fori_loop(0, 8, lgloop, 0)
            pltpu.async_copy(rbuf.at[:, pl.ds(0, 1)],
                             outn_hbm.at[:, pl.ds(nb + p, 1)], sem)
        return carry

    lax.fori_loop(0, N_PAIRS, node_pair, 0)
    for rbuf, sem in bufs:
        pltpu.make_async_copy(
            outn_hbm.at[:, pl.ds(0, 1)], rbuf.at[:, pl.ds(0, 1)], sem).wait()

    edge_b0 = jnp.minimum(wid * EB_PER_W, EB_TOT - EB_PER_W)

    def edge_pair(k, carry):
        bb = edge_b0 + k * 4
        eds = [pltpu.async_copy(
                   ea_hbm.at[pl.ds(c * N_EDGES + bb * 128, 512)],
                   ebuf.at[pl.ds(c * 512, 512)], si) for c in range(3)]
        for d in eds:
            d.wait()
        for p, (rbuf, sem) in enumerate(bufs):
            @pl.when(k > 0)
            def _():
                pltpu.make_async_copy(
                    oute_hbm.at[:, pl.ds(0, 2)], rbuf, sem).wait()
            for b in range(2):
                def lgloop(lg, c2, b=b):
                    soff = pl.multiple_of(lg * 16, 16)
                    q = soff + (p * 2 + b) * 128
                    e0 = ebuf[pl.ds(q, 16)]
                    e1 = ebuf[pl.ds(q + 512, 16)]
                    e2 = ebuf[pl.ds(q + 1024, 16)]
                    a0 = ((e0 * 6 + e1) * 2 + e2) * STR
                    for t in range(64):
                        vals = plsc.load_gather(fev, [a0 + t])
                        rbuf[t // 8, b, t % 8, pl.ds(soff, 16)] = vals
                    return c2

                lax.fori_loop(0, 8, lgloop, 0)
            pltpu.async_copy(rbuf, oute_hbm.at[:, pl.ds(bb + p * 2, 2)], sem)
        return carry

    lax.fori_loop(0, E_PAIRS, edge_pair, 0)
    for rbuf, sem in bufs:
        pltpu.make_async_copy(oute_hbm.at[:, pl.ds(0, 2)], rbuf, sem).wait()


_sc_call = pl.kernel(
    _sc_body,
    out_type=(
        jax.ShapeDtypeStruct((8, EB_TOT, 8, 128), jnp.float32),
        jax.ShapeDtypeStruct((8, NB_TOT, 8, 128), jnp.float32),
    ),
    mesh=plsc.VectorSubcoreMesh(core_axis_name="c", subcore_axis_name="s"),
    compiler_params=pltpu.CompilerParams(
        needs_layout_passes=False, use_tc_tiling_on_sc=True),
    scratch_types=[
        pltpu.VMEM((S0,), jnp.float32),
        pltpu.VMEM((SB,), jnp.float32),
        pltpu.VMEM((SC_,), jnp.float32),
        pltpu.VMEM((SD,), jnp.float32),
        pltpu.VMEM((SE,), jnp.float32),
        pltpu.VMEM((512 * 3,), jnp.int32),
        pltpu.VMEM((256 * 9,), jnp.int32),
        pltpu.VMEM((8, 2, 8, 128), jnp.float32),
        pltpu.VMEM((8, 2, 8, 128), jnp.float32),
        pltpu.SemaphoreType.DMA,
        pltpu.SemaphoreType.DMA,
        pltpu.SemaphoreType.DMA,
    ],
)


def kernel(x, edge_attr, atom_tables, bond_tables):
    x32 = x.astype(jnp.int32)
    xf = jnp.pad(x32, ((0, N_PAD - N_NODES), (0, 0))).T.reshape(-1)
    eaf = edge_attr.astype(jnp.int32).T.reshape(-1)
    t = [a.astype(jnp.float32) for a in atom_tables]
    bo = [a.astype(jnp.float32) for a in bond_tables]
    def _flat(tbl2d, size):
        r = tbl2d.shape[0]
        f = jnp.pad(tbl2d, ((0, 0), (0, STR - HIDDEN))).reshape(-1)
        return jnp.pad(f, (0, size - r * STR))

    t0f = _flat(t[0], S0)
    tbf = _flat((t[1][:, None, None] + t[2][None, :, None]
                 + t[3][None, None, :]).reshape(RB, HIDDEN), SB)
    tcf = _flat((t[4][:, None, None] + t[5][None, :, None]
                 + t[6][None, None, :]).reshape(RC, HIDDEN), SC_)
    tdf = _flat((t[7][:, None] + t[8][None, :]).reshape(RD, HIDDEN), SD)
    fef = _flat((bo[0][:, None, None] + bo[1][None, :, None]
                 + bo[2][None, None, :]).reshape(RE, HIDDEN), SE)
    o4e, o4n = _sc_call(xf, eaf, t0f, tbf, tcf, tdf, fef)
    edge_emb = o4e.transpose(1, 3, 0, 2).reshape(N_EDGES, HIDDEN)
    node_emb = o4n.transpose(1, 3, 0, 2).reshape(N_PAD, HIDDEN)[:N_NODES]
    return (node_emb, edge_emb)


# speedup vs baseline: 10.6740x; 1.5212x over previous
import jax
import jax.numpy as jnp
from jax import lax
from jax.experimental import pallas as pl
from jax.experimental.pallas import tpu as pltpu
from jax.experimental.pallas import tpu_sc as plsc

HIDDEN = 64
N_NODES = 50000
N_EDGES = 800000

NC = 2
NS = 16
NW = NC * NS
L = 16

EB_TOT = N_EDGES // 128
EB_PER_W = 196
E_PAIRS = EB_PER_W // 4

N_PAD = 50048
NB_TOT = N_PAD // 128
NB_PER_W = 14
N_PAIRS = NB_PER_W // 2

R0, RB, RC, RD, RE = 119, 576, 360, 4, 60
STR = 65


def _padup(n):
    return (n + 127) // 128 * 128


S0, SB, SC_, SD, SE = (_padup(r * STR) for r in (R0, RB, RC, RD, RE))


def _sc_body(x_hbm, ea_hbm, t0_hbm, tb_hbm, tc_hbm, td_hbm, fe_hbm,
             oute_hbm, outn_hbm,
             t0v, tbv, tcv, tdv, fev, ebuf, xbuf, r0buf, r1buf, s0, s1, si):
    cid = lax.axis_index("c")
    sid = lax.axis_index("s")
    wid = sid * NC + cid

    pltpu.sync_copy(t0_hbm, t0v)
    pltpu.sync_copy(tb_hbm, tbv)
    pltpu.sync_copy(tc_hbm, tcv)
    pltpu.sync_copy(td_hbm, tdv)
    pltpu.sync_copy(fe_hbm, fev)

    bufs = ((r0buf, s0), (r1buf, s1))

    node_b0 = jnp.minimum(wid * (NB_PER_W - 1), NB_TOT - NB_PER_W)

    def node_pair(i, carry):
        nb = node_b0 + i * 2
        xds = [pltpu.async_copy(
                   x_hbm.at[pl.ds(j * N_PAD + nb * 128, 256)],
                   xbuf.at[pl.ds(j * 256, 256)], si) for j in range(9)]
        for d in xds:
            d.wait()
        for p, (rbuf, sem) in enumerate(bufs):
            @pl.when(i > 0)
            def _():
                pltpu.make_async_copy(
                    outn_hbm.at[:, pl.ds(0, 1)], rbuf.at[:, pl.ds(0, 1)],
                    sem).wait()

            @plsc.parallel_loop(0, 8)
            def lgloop(lg, p=p, rbuf=rbuf):
                soff = pl.multiple_of(lg * 16, 16)
                xc = [xbuf[pl.ds(soff + (j * 256 + p * 128), 16)]
                      for j in range(9)]
                aa = xc[0] * STR
                ab = ((xc[1] * 12 + xc[2]) * 12 + xc[3]) * STR
                ac = ((xc[4] * 6 + xc[5]) * 6 + xc[6]) * STR
                ad = (xc[7] * 2 + xc[8]) * STR
                for t in range(64):
                    va = plsc.load_gather(t0v, [aa + t])
                    vb = plsc.load_gather(tbv, [ab + t])
                    vc = plsc.load_gather(tcv, [ac + t])
                    vd = plsc.load_gather(tdv, [ad + t])
                    rbuf[t // 8, 0, t % 8, pl.ds(soff, 16)] = (
                        (va + vb) + (vc + vd))

            pltpu.async_copy(rbuf.at[:, pl.ds(0, 1)],
                             outn_hbm.at[:, pl.ds(nb + p, 1)], sem)
        return carry

    lax.fori_loop(0, N_PAIRS, node_pair, 0)
    for rbuf, sem in bufs:
        pltpu.make_async_copy(
            outn_hbm.at[:, pl.ds(0, 1)], rbuf.at[:, pl.ds(0, 1)], sem).wait()

    edge_b0 = jnp.minimum(wid * EB_PER_W, EB_TOT - EB_PER_W)

    def edge_pair(k, carry):
        bb = edge_b0 + k * 4
        eds = [pltpu.async_copy(
                   ea_hbm.at[pl.ds(c * N_EDGES + bb * 128, 512)],
                   ebuf.at[pl.ds(c * 512, 512)], si) for c in range(3)]
        for d in eds:
            d.wait()
        for p, (rbuf, sem) in enumerate(bufs):
            @pl.when(k > 0)
            def _():
                pltpu.make_async_copy(
                    oute_hbm.at[:, pl.ds(0, 2)], rbuf, sem).wait()
            for b in range(2):
                @plsc.parallel_loop(0, 8)
                def lgloop(lg, b=b, p=p, rbuf=rbuf):
                    soff = pl.multiple_of(lg * 16, 16)
                    q = soff + (p * 2 + b) * 128
                    e0 = ebuf[pl.ds(q, 16)]
                    e1 = ebuf[pl.ds(q + 512, 16)]
                    e2 = ebuf[pl.ds(q + 1024, 16)]
                    a0 = ((e0 * 6 + e1) * 2 + e2) * STR
                    for t in range(64):
                        vals = plsc.load_gather(fev, [a0 + t])
                        rbuf[t // 8, b, t % 8, pl.ds(soff, 16)] = vals
            pltpu.async_copy(rbuf, oute_hbm.at[:, pl.ds(bb + p * 2, 2)], sem)
        return carry

    lax.fori_loop(0, E_PAIRS, edge_pair, 0)
    for rbuf, sem in bufs:
        pltpu.make_async_copy(oute_hbm.at[:, pl.ds(0, 2)], rbuf, sem).wait()


_sc_call = pl.kernel(
    _sc_body,
    out_type=(
        jax.ShapeDtypeStruct((8, EB_TOT, 8, 128), jnp.float32),
        jax.ShapeDtypeStruct((8, NB_TOT, 8, 128), jnp.float32),
    ),
    mesh=plsc.VectorSubcoreMesh(core_axis_name="c", subcore_axis_name="s"),
    compiler_params=pltpu.CompilerParams(
        needs_layout_passes=False, use_tc_tiling_on_sc=True),
    scratch_types=[
        pltpu.VMEM((S0,), jnp.float32),
        pltpu.VMEM((SB,), jnp.float32),
        pltpu.VMEM((SC_,), jnp.float32),
        pltpu.VMEM((SD,), jnp.float32),
        pltpu.VMEM((SE,), jnp.float32),
        pltpu.VMEM((512 * 3,), jnp.int32),
        pltpu.VMEM((256 * 9,), jnp.int32),
        pltpu.VMEM((8, 2, 8, 128), jnp.float32),
        pltpu.VMEM((8, 2, 8, 128), jnp.float32),
        pltpu.SemaphoreType.DMA,
        pltpu.SemaphoreType.DMA,
        pltpu.SemaphoreType.DMA,
    ],
)


def kernel(x, edge_attr, atom_tables, bond_tables):
    x32 = x.astype(jnp.int32)
    xf = jnp.pad(x32, ((0, N_PAD - N_NODES), (0, 0))).T.reshape(-1)
    eaf = edge_attr.astype(jnp.int32).T.reshape(-1)
    t = [a.astype(jnp.float32) for a in atom_tables]
    bo = [a.astype(jnp.float32) for a in bond_tables]
    def _flat(tbl2d, size):
        r = tbl2d.shape[0]
        f = jnp.pad(tbl2d, ((0, 0), (0, STR - HIDDEN))).reshape(-1)
        return jnp.pad(f, (0, size - r * STR))

    t0f = _flat(t[0], S0)
    tbf = _flat((t[1][:, None, None] + t[2][None, :, None]
                 + t[3][None, None, :]).reshape(RB, HIDDEN), SB)
    tcf = _flat((t[4][:, None, None] + t[5][None, :, None]
                 + t[6][None, None, :]).reshape(RC, HIDDEN), SC_)
    tdf = _flat((t[7][:, None] + t[8][None, :]).reshape(RD, HIDDEN), SD)
    fef = _flat((bo[0][:, None, None] + bo[1][None, :, None]
                 + bo[2][None, None, :]).reshape(RE, HIDDEN), SE)
    o4e, o4n = _sc_call(xf, eaf, t0f, tbf, tcf, tdf, fef)
    edge_emb = o4e.transpose(1, 3, 0, 2).reshape(N_EDGES, HIDDEN)
    node_emb = o4n.transpose(1, 3, 0, 2).reshape(N_PAD, HIDDEN)[:N_NODES]
    return (node_emb, edge_emb)


# speedup vs baseline: 12.5588x; 1.1766x over previous
import jax
import jax.numpy as jnp
from jax import lax
from jax.experimental import pallas as pl
from jax.experimental.pallas import tpu as pltpu
from jax.experimental.pallas import tpu_sc as plsc

HIDDEN = 64
N_NODES = 50000
N_EDGES = 800000

NC = 2
NS = 16
NW = NC * NS
L = 16

EB_TOT = N_EDGES // 128
EB_PER_W = 196
E_QUADS = EB_PER_W // 4

N_PAD = 50048
NB_TOT = N_PAD // 128
NB_PER_W = 14
N_PAIRS = NB_PER_W // 2

R0, RB, RC, RD, RE = 119, 576, 360, 4, 60
STR = 65


def _padup(n):
    return (n + 127) // 128 * 128


S0, SB, SC_, SD, SE = (_padup(r * STR) for r in (R0, RB, RC, RD, RE))


def _sc_body(x_hbm, ea_hbm, t0_hbm, tb_hbm, tc_hbm, td_hbm, fe_hbm,
             oute_hbm, outn_hbm,
             t0v, tbv, tcv, tdv, fev, ebuf, xbuf,
             rb0, rb1, rb2, rb3, s0, s1, s2, s3, si):
    cid = lax.axis_index("c")
    sid = lax.axis_index("s")
    wid = sid * NC + cid

    pltpu.sync_copy(t0_hbm, t0v)
    pltpu.sync_copy(tb_hbm, tbv)
    pltpu.sync_copy(tc_hbm, tcv)
    pltpu.sync_copy(td_hbm, tdv)
    pltpu.sync_copy(fe_hbm, fev)

    bufs4 = ((rb0, s0), (rb1, s1), (rb2, s2), (rb3, s3))

    node_b0 = jnp.minimum(wid * (NB_PER_W - 1), NB_TOT - NB_PER_W)

    def node_pair(i, carry):
        nb = node_b0 + i * 2
        xds = [pltpu.async_copy(
                   x_hbm.at[pl.ds(j * N_PAD + nb * 128, 256)],
                   xbuf.at[pl.ds(j * 256, 256)], si) for j in range(9)]
        for d in xds:
            d.wait()
        for p, (rbuf, sem) in enumerate(bufs4[:2]):
            @pl.when(i > 0)
            def _():
                pltpu.make_async_copy(
                    outn_hbm.at[:, pl.ds(0, 1)], rbuf, sem).wait()

            @plsc.parallel_loop(0, 8)
            def lgloop(lg, p=p, rbuf=rbuf):
                soff = pl.multiple_of(lg * 16, 16)
                xc = [xbuf[pl.ds(soff + (j * 256 + p * 128), 16)]
                      for j in range(9)]
                aa = xc[0] * STR
                ab = ((xc[1] * 12 + xc[2]) * 12 + xc[3]) * STR
                ac = ((xc[4] * 6 + xc[5]) * 6 + xc[6]) * STR
                ad = (xc[7] * 2 + xc[8]) * STR
                for t in range(64):
                    va = plsc.load_gather(t0v, [aa + t])
                    vb = plsc.load_gather(tbv, [ab + t])
                    vc = plsc.load_gather(tcv, [ac + t])
                    vd = plsc.load_gather(tdv, [ad + t])
                    rbuf[t // 8, 0, t % 8, pl.ds(soff, 16)] = (
                        (va + vb) + (vc + vd))

            pltpu.async_copy(rbuf, outn_hbm.at[:, pl.ds(nb + p, 1)], sem)
        return carry

    lax.fori_loop(0, N_PAIRS, node_pair, 0)
    for rbuf, sem in bufs4[:2]:
        pltpu.make_async_copy(
            outn_hbm.at[:, pl.ds(0, 1)], rbuf, sem).wait()

    edge_b0 = jnp.minimum(wid * EB_PER_W, EB_TOT - EB_PER_W)

    def _eseg_start(k, slot):
        bb = edge_b0 + k * 4
        for c in range(3):
            pltpu.async_copy(
                ea_hbm.at[pl.ds(c * N_EDGES + bb * 128, 512)],
                ebuf.at[pl.ds(slot * 1536 + c * 512, 512)], si)

    def _eseg_drain():
        for c in range(3):
            pltpu.make_async_copy(
                ea_hbm.at[pl.ds(0, 512)], ebuf.at[pl.ds(0, 512)], si).wait()

    _eseg_start(0, 0)

    def edge_quad(k, carry):
        bb = edge_b0 + k * 4
        _eseg_drain()
        _eseg_start(jnp.minimum(k + 1, E_QUADS - 1), (k + 1) % 2)
        sb = pl.multiple_of((k % 2) * 1536, 16)
        for p, (rbuf, sem) in enumerate(bufs4):
            @pl.when(k > 0)
            def _():
                pltpu.make_async_copy(
                    oute_hbm.at[:, pl.ds(0, 1)], rbuf, sem).wait()

            @plsc.parallel_loop(0, 8)
            def lgloop(lg, p=p, rbuf=rbuf, sb=sb):
                soff = pl.multiple_of(lg * 16, 16)
                q = sb + soff + p * 128
                e0 = ebuf[pl.ds(q, 16)]
                e1 = ebuf[pl.ds(q + 512, 16)]
                e2 = ebuf[pl.ds(q + 1024, 16)]
                a0 = ((e0 * 6 + e1) * 2 + e2) * STR
                for t in range(64):
                    vals = plsc.load_gather(fev, [a0 + t])
                    rbuf[t // 8, 0, t % 8, pl.ds(soff, 16)] = vals

            pltpu.async_copy(rbuf, oute_hbm.at[:, pl.ds(bb + p, 1)], sem)
        return carry

    lax.fori_loop(0, E_QUADS, edge_quad, 0)
    _eseg_drain()
    for rbuf, sem in bufs4:
        pltpu.make_async_copy(oute_hbm.at[:, pl.ds(0, 1)], rbuf, sem).wait()


_sc_call = pl.kernel(
    _sc_body,
    out_type=(
        jax.ShapeDtypeStruct((8, EB_TOT, 8, 128), jnp.float32),
        jax.ShapeDtypeStruct((8, NB_TOT, 8, 128), jnp.float32),
    ),
    mesh=plsc.VectorSubcoreMesh(core_axis_name="c", subcore_axis_name="s"),
    compiler_params=pltpu.CompilerParams(
        needs_layout_passes=False, use_tc_tiling_on_sc=True),
    scratch_types=[
        pltpu.VMEM((S0,), jnp.float32),
        pltpu.VMEM((SB,), jnp.float32),
        pltpu.VMEM((SC_,), jnp.float32),
        pltpu.VMEM((SD,), jnp.float32),
        pltpu.VMEM((SE,), jnp.float32),
        pltpu.VMEM((2 * 512 * 3,), jnp.int32),
        pltpu.VMEM((256 * 9,), jnp.int32),
        pltpu.VMEM((8, 1, 8, 128), jnp.float32),
        pltpu.VMEM((8, 1, 8, 128), jnp.float32),
        pltpu.VMEM((8, 1, 8, 128), jnp.float32),
        pltpu.VMEM((8, 1, 8, 128), jnp.float32),
        pltpu.SemaphoreType.DMA,
        pltpu.SemaphoreType.DMA,
        pltpu.SemaphoreType.DMA,
        pltpu.SemaphoreType.DMA,
        pltpu.SemaphoreType.DMA,
    ],
)


def kernel(x, edge_attr, atom_tables, bond_tables):
    x32 = x.astype(jnp.int32)
    xf = jnp.pad(x32, ((0, N_PAD - N_NODES), (0, 0))).T.reshape(-1)
    eaf = edge_attr.astype(jnp.int32).T.reshape(-1)
    t = [a.astype(jnp.float32) for a in atom_tables]
    bo = [a.astype(jnp.float32) for a in bond_tables]
    def _flat(tbl2d, size):
        r = tbl2d.shape[0]
        f = jnp.pad(tbl2d, ((0, 0), (0, STR - HIDDEN))).reshape(-1)
        return jnp.pad(f, (0, size - r * STR))

    t0f = _flat(t[0], S0)
    tbf = _flat((t[1][:, None, None] + t[2][None, :, None]
                 + t[3][None, None, :]).reshape(RB, HIDDEN), SB)
    tcf = _flat((t[4][:, None, None] + t[5][None, :, None]
                 + t[6][None, None, :]).reshape(RC, HIDDEN), SC_)
    tdf = _flat((t[7][:, None] + t[8][None, :]).reshape(RD, HIDDEN), SD)
    fef = _flat((bo[0][:, None, None] + bo[1][None, :, None]
                 + bo[2][None, None, :]).reshape(RE, HIDDEN), SE)
    o4e, o4n = _sc_call(xf, eaf, t0f, tbf, tcf, tdf, fef)
    edge_emb = o4e.transpose(1, 3, 0, 2).reshape(N_EDGES, HIDDEN)
    node_emb = o4n.transpose(1, 3, 0, 2).reshape(N_PAD, HIDDEN)[:N_NODES]
    return (node_emb, edge_emb)
